# baseline (speedup 1.0000x reference)
import functools

import jax
import jax.numpy as jnp
from jax import lax
from jax.experimental import pallas as pl
from jax.experimental.pallas import tpu as pltpu
from jax.experimental.pallas import tpu_sc as plsc

N_ITEM = 1000000
DIM = 64
BATCH = 4096
HIST = 200

_TOTAL = BATCH * HIST
_CHUNK = 128
_NUM_ROWS = _TOTAL // _CHUNK
_NW = 32
_ROWS_PER_W = _NUM_ROWS // _NW
_NBUF = 4


def _make_sc_lookup():
    mesh = plsc.VectorSubcoreMesh(core_axis_name="c", subcore_axis_name="s")

    @functools.partial(
        pl.kernel,
        out_type=jax.ShapeDtypeStruct((_NUM_ROWS, _CHUNK, 2 * DIM), jnp.float32),
        mesh=mesh,
        scratch_types=[
            pltpu.VMEM((_ROWS_PER_W, _CHUNK), jnp.int32),
            pltpu.VMEM((_NBUF, _CHUNK, 2 * DIM), jnp.float32),
        ]
        + [pltpu.SemaphoreType.DMA] * (2 * _NBUF),
    )
    def body(idx_hbm, tab_hbm, out_hbm, idx_v, rows_v, *sems):
        gsem = sems[:_NBUF]
        wsem = sems[_NBUF:]
        wid = lax.axis_index("s") * 2 + lax.axis_index("c")
        row0 = wid * _ROWS_PER_W
        pltpu.sync_copy(idx_hbm.at[pl.ds(row0, _ROWS_PER_W)], idx_v)

        def fire(j, b):
            pltpu.async_copy(tab_hbm.at[idx_v.at[j]], rows_v.at[b], gsem[b])

        def gwait(b):
            pltpu.make_async_copy(
                tab_hbm.at[pl.ds(0, _CHUNK)], rows_v.at[b], gsem[b]).wait()

        def wstart(j, b):
            pltpu.async_copy(rows_v.at[b], out_hbm.at[row0 + j], wsem[b])

        def wwait(b):
            pltpu.make_async_copy(
                rows_v.at[b], out_hbm.at[0], wsem[b]).wait()

        for b in range(_NBUF):
            fire(b, b)

        def outer(g, _):
            base = g * _NBUF
            for b in range(_NBUF):
                j = base + b
                gwait(b)
                wstart(j, b)
                wwait(b)
                fire(j + _NBUF, b)
            return 0

        lax.fori_loop(0, _ROWS_PER_W // _NBUF - 1, outer, 0)

        base = _ROWS_PER_W - _NBUF
        for b in range(_NBUF):
            gwait(b)
            wstart(base + b, b)
        for b in range(_NBUF):
            wwait(b)

    return body


_sc_lookup = _make_sc_lookup()


@jax.jit
def kernel(indices, language_table, id_table):
    half = lax.optimization_barrier(
        jnp.pad(language_table, ((0, 0), (0, DIM))))
    table = lax.dynamic_update_slice(half, id_table, (0, DIM))
    idx = indices.astype(jnp.int32).reshape(_NUM_ROWS, _CHUNK)
    out = _sc_lookup(idx, table)
    return out.reshape(BATCH, HIST, 2 * DIM)


# speedup vs baseline: 3.2900x; 3.2900x over previous
import functools

import jax
import jax.numpy as jnp
from jax import lax
from jax.experimental import pallas as pl
from jax.experimental.pallas import tpu as pltpu
from jax.experimental.pallas import tpu_sc as plsc

N_ITEM = 1000000
DIM = 64
BATCH = 4096
HIST = 200

_TOTAL = BATCH * HIST
_CHUNK = 128
_NUM_ROWS = _TOTAL // _CHUNK
_NW = 32
_ROWS_PER_W = _NUM_ROWS // _NW
_NBUF = 5


def _make_sc_lookup():
    mesh = plsc.VectorSubcoreMesh(core_axis_name="c", subcore_axis_name="s")

    @functools.partial(
        pl.kernel,
        out_type=jax.ShapeDtypeStruct((_NUM_ROWS, _CHUNK, 2 * DIM), jnp.float32),
        mesh=mesh,
        scratch_types=[
            pltpu.VMEM((_ROWS_PER_W, _CHUNK), jnp.int32),
            pltpu.VMEM((_NBUF, _CHUNK, 2 * DIM), jnp.float32),
        ]
        + [pltpu.SemaphoreType.DMA] * (2 * _NBUF),
    )
    def body(idx_hbm, tab_hbm, out_hbm, idx_v, rows_v, *sems):
        gsem = sems[:_NBUF]
        wsem = sems[_NBUF:]
        wid = lax.axis_index("s") * 2 + lax.axis_index("c")
        row0 = wid * _ROWS_PER_W
        pltpu.sync_copy(idx_hbm.at[pl.ds(row0, _ROWS_PER_W)], idx_v)

        def fire(j, b):
            pltpu.async_copy(tab_hbm.at[idx_v.at[j]], rows_v.at[b], gsem[b])

        def gwait(b):
            pltpu.make_async_copy(
                tab_hbm.at[pl.ds(0, _CHUNK)], rows_v.at[b], gsem[b]).wait()

        def wstart(j, b):
            pltpu.async_copy(rows_v.at[b], out_hbm.at[row0 + j], wsem[b])

        def wwait(b):
            pltpu.make_async_copy(
                rows_v.at[b], out_hbm.at[0], wsem[b]).wait()

        for b in range(_NBUF):
            fire(b, b)

        def outer(g, _):
            base = g * _NBUF
            for b in range(_NBUF):
                j = base + b
                gwait(b)
                wstart(j, b)
                wwait(b)
                fire(j + _NBUF, b)
            return 0

        lax.fori_loop(0, _ROWS_PER_W // _NBUF - 1, outer, 0)

        base = _ROWS_PER_W - _NBUF
        for b in range(_NBUF):
            gwait(b)
            wstart(base + b, b)
        for b in range(_NBUF):
            wwait(b)

    return body


_sc_lookup = _make_sc_lookup()


@jax.jit
def kernel(indices, language_table, id_table):
    table = jnp.concatenate([language_table, id_table], axis=1)
    idx = indices.astype(jnp.int32).reshape(_NUM_ROWS, _CHUNK)
    out = _sc_lookup(idx, table)
    return out.reshape(BATCH, HIST, 2 * DIM)
